# baseline (speedup 1.0000x reference)
import functools

import jax
import jax.numpy as jnp
from jax import lax
from jax.experimental import pallas as pl
from jax.experimental.pallas import tpu as pltpu
from jax.experimental.pallas import tpu_sc as plsc

_NUM_EMBEDDINGS = 100001
_EMBED_DIM = 96
_BATCH = 16384


@functools.lru_cache(maxsize=None)
def _build_sc_gather():
    info = plsc.get_sparse_core_info()
    nc, ns = info.num_cores, info.num_subcores
    nw = nc * ns
    assert _BATCH % (8 * nw) == 0
    b_per_w = _BATCH // nw

    mesh = plsc.VectorSubcoreMesh(core_axis_name="c", subcore_axis_name="s")

    @functools.partial(
        pl.kernel,
        mesh=mesh,
        out_type=jax.ShapeDtypeStruct((_BATCH, _EMBED_DIM), jnp.float32),
        scratch_types=[
            pltpu.VMEM((b_per_w,), jnp.int32),
            pltpu.VMEM((b_per_w, _EMBED_DIM), jnp.float32),
            pltpu.SemaphoreType.DMA,
        ],
        compiler_params=pltpu.CompilerParams(use_tc_tiling_on_sc=False),
    )
    def gather_kernel(idx_hbm, table_hbm, out_hbm, idx_v, rows_v, sem):
        wid = lax.axis_index("s") * nc + lax.axis_index("c")
        base = wid * b_per_w
        pltpu.sync_copy(idx_hbm.at[pl.ds(base, b_per_w)], idx_v)
        pltpu.async_copy(table_hbm.at[idx_v], rows_v, sem).wait()
        pltpu.sync_copy(rows_v, out_hbm.at[pl.ds(base, b_per_w)])

    return gather_kernel


@jax.jit
def kernel(viewer_indices, embedding_table):
    gather = _build_sc_gather()
    return gather(viewer_indices.astype(jnp.int32), embedding_table)


# speedup vs baseline: 3.5565x; 3.5565x over previous
import functools

import jax
import jax.numpy as jnp
from jax import lax
from jax.experimental import pallas as pl
from jax.experimental.pallas import tpu as pltpu
from jax.experimental.pallas import tpu_sc as plsc

_NUM_EMBEDDINGS = 100001
_EMBED_DIM = 96
_BATCH = 16384
_UNROLL = 8


@functools.lru_cache(maxsize=None)
def _build_sc_gather():
    info = plsc.get_sparse_core_info()
    nc, ns = info.num_cores, info.num_subcores
    nw = nc * ns
    assert _BATCH % (8 * nw) == 0
    b_per_w = _BATCH // nw

    mesh = plsc.VectorSubcoreMesh(core_axis_name="c", subcore_axis_name="s")

    @functools.partial(
        pl.kernel,
        mesh=mesh,
        out_type=jax.ShapeDtypeStruct((_BATCH, _EMBED_DIM), jnp.float32),
        scratch_types=[
            pltpu.VMEM((b_per_w,), jnp.int32),
            pltpu.VMEM((b_per_w, _EMBED_DIM), jnp.float32),
            pltpu.SemaphoreType.DMA,
        ],
    )
    def gather_kernel(idx_hbm, table_hbm, out_hbm, idx_v, rows_v, sem):
        wid = lax.axis_index("s") * nc + lax.axis_index("c")
        base = wid * b_per_w
        pltpu.sync_copy(idx_hbm.at[pl.ds(base, b_per_w)], idx_v)

        def fire(c, carry):
            b0 = c * 16
            vec = idx_v[pl.ds(b0, 16)]
            for j in range(16):
                r = vec[j]
                pltpu.make_async_copy(
                    table_hbm.at[r], rows_v.at[b0 + j], sem
                ).start()
            return carry

        lax.fori_loop(0, b_per_w // 16, fire, 0, unroll=False)
        pltpu.make_async_copy(
            table_hbm.at[pl.ds(0, b_per_w)], rows_v, sem
        ).wait()
        pltpu.sync_copy(rows_v, out_hbm.at[pl.ds(base, b_per_w)])

    return gather_kernel


@jax.jit
def kernel(viewer_indices, embedding_table):
    gather = _build_sc_gather()
    return gather(viewer_indices.astype(jnp.int32), embedding_table)


# speedup vs baseline: 3.7983x; 1.0680x over previous
import functools

import jax
import jax.numpy as jnp
from jax import lax
from jax.experimental import pallas as pl
from jax.experimental.pallas import tpu as pltpu
from jax.experimental.pallas import tpu_sc as plsc

_NUM_EMBEDDINGS = 100001
_EMBED_DIM = 96
_BATCH = 16384
_CHUNK = 4096


@functools.lru_cache(maxsize=None)
def _build_sc_gather():
    info = plsc.get_sparse_core_info()
    nc, ns = info.num_cores, info.num_subcores
    nw = nc * ns
    rows_per_w = _EMBED_DIM // nw
    assert _EMBED_DIM % nw == 0 and _BATCH % _CHUNK == 0

    mesh = plsc.VectorSubcoreMesh(core_axis_name="c", subcore_axis_name="s")

    @functools.partial(
        pl.kernel,
        mesh=mesh,
        out_type=jax.ShapeDtypeStruct((_EMBED_DIM, _BATCH), jnp.float32),
        scratch_types=[
            pltpu.VMEM((_BATCH,), jnp.int32),
            pltpu.VMEM((1, _NUM_EMBEDDINGS), jnp.float32),
            pltpu.VMEM((2, _CHUNK), jnp.float32),
            pltpu.SemaphoreType.DMA,
            pltpu.SemaphoreType.DMA,
        ],
        compiler_params=pltpu.CompilerParams(needs_layout_passes=False),
    )
    def gather_kernel(idx_hbm, tab_t_hbm, out_t_hbm, idx_v, row_v, stage_v,
                      row_sem, out_sem):
        wid = lax.axis_index("s") * nc + lax.axis_index("c")
        pltpu.sync_copy(idx_hbm, idx_v)
        zero_v = jnp.zeros((16,), jnp.int32)

        for k in range(rows_per_w):
            c = wid * rows_per_w + k
            pltpu.async_copy(
                tab_t_hbm.at[pl.ds(c, 1), :], row_v, row_sem
            ).wait()

            for h in range(_BATCH // _CHUNK):
                buf = h % 2
                if h >= 2:
                    pltpu.make_async_copy(
                        stage_v.at[buf], out_t_hbm.at[c, pl.ds(0, _CHUNK)],
                        out_sem,
                    ).wait()

                def gather_vec(i, carry, h=h, buf=buf):
                    vec = idx_v[pl.ds(h * _CHUNK + i * 16, 16)]
                    g = plsc.load_gather(row_v, [zero_v, vec])
                    stage_v[buf, pl.ds(i * 16, 16)] = g
                    return carry

                lax.fori_loop(0, _CHUNK // 16, gather_vec, 0, unroll=4)
                pltpu.make_async_copy(
                    stage_v.at[buf],
                    out_t_hbm.at[c, pl.ds(h * _CHUNK, _CHUNK)],
                    out_sem,
                ).start()

            for _ in range(2):
                pltpu.make_async_copy(
                    stage_v.at[0], out_t_hbm.at[c, pl.ds(0, _CHUNK)], out_sem
                ).wait()

    return gather_kernel


@jax.jit
def kernel(viewer_indices, embedding_table):
    gather = _build_sc_gather()
    out_t = gather(viewer_indices.astype(jnp.int32), embedding_table.T)
    return out_t.T


# speedup vs baseline: 5.5326x; 1.4566x over previous
import functools

import jax
import jax.numpy as jnp
from jax import lax
from jax.experimental import pallas as pl
from jax.experimental.pallas import tpu as pltpu
from jax.experimental.pallas import tpu_sc as plsc

_NUM_EMBEDDINGS = 100001
_EMBED_DIM = 96
_BATCH = 16384
_CHUNK = 4096


@functools.lru_cache(maxsize=None)
def _build_sc_gather():
    info = plsc.get_sparse_core_info()
    nc, ns = info.num_cores, info.num_subcores
    nw = nc * ns
    rows_per_w = _EMBED_DIM // nw
    assert _EMBED_DIM % nw == 0 and _BATCH % _CHUNK == 0

    mesh = plsc.VectorSubcoreMesh(core_axis_name="c", subcore_axis_name="s")

    @functools.partial(
        pl.kernel,
        mesh=mesh,
        out_type=jax.ShapeDtypeStruct((_EMBED_DIM, _BATCH), jnp.float32),
        scratch_types=[
            pltpu.VMEM((_BATCH,), jnp.int32),
            pltpu.VMEM((1, _NUM_EMBEDDINGS), jnp.float32),
            pltpu.VMEM((2, _CHUNK), jnp.float32),
            pltpu.SemaphoreType.DMA,
            pltpu.SemaphoreType.DMA,
        ],
        compiler_params=pltpu.CompilerParams(needs_layout_passes=False),
    )
    def gather_kernel(idx_hbm, tab_t_hbm, out_t_hbm, idx_v, row_v, stage_v,
                      row_sem, out_sem):
        wid = lax.axis_index("s") * nc + lax.axis_index("c")
        pltpu.sync_copy(idx_hbm, idx_v)
        zero_v = jnp.zeros((16,), jnp.int32)

        for k in range(rows_per_w):
            c = wid * rows_per_w + k
            pltpu.async_copy(
                tab_t_hbm.at[pl.ds(c, 1), :], row_v, row_sem
            ).wait()

            for h in range(_BATCH // _CHUNK):
                buf = h % 2
                if h >= 2:
                    pltpu.make_async_copy(
                        stage_v.at[buf], out_t_hbm.at[c, pl.ds(0, _CHUNK)],
                        out_sem,
                    ).wait()

                def gather_group(i, carry, h=h, buf=buf):
                    vecs = [
                        idx_v[pl.ds(h * _CHUNK + (i * 8 + j) * 16, 16)]
                        for j in range(8)
                    ]
                    gs = [plsc.load_gather(row_v, [zero_v, v]) for v in vecs]
                    for j, g in enumerate(gs):
                        stage_v[buf, pl.ds((i * 8 + j) * 16, 16)] = g
                    return carry

                lax.fori_loop(0, _CHUNK // 128, gather_group, 0, unroll=1)
                pltpu.make_async_copy(
                    stage_v.at[buf],
                    out_t_hbm.at[c, pl.ds(h * _CHUNK, _CHUNK)],
                    out_sem,
                ).start()

            for _ in range(2):
                pltpu.make_async_copy(
                    stage_v.at[0], out_t_hbm.at[c, pl.ds(0, _CHUNK)], out_sem
                ).wait()

    return gather_kernel


@jax.jit
def kernel(viewer_indices, embedding_table):
    gather = _build_sc_gather()
    out_t = gather(viewer_indices.astype(jnp.int32), embedding_table.T)
    return out_t.T
